# baseline (speedup 1.0000x reference)
import jax
import jax.numpy as jnp
from jax.experimental import pallas as pl
from jax.experimental.pallas import tpu as pltpu

NUM_EXPERTS = 8
TOP_K = 2
HIDDEN = 1024
INTER = 512
TOKENS = 2048

_NEG = -1e30


def _route(x, gw, bias):
    logits = jax.lax.dot_general(
        x, gw, (((1,), (1,)), ((), ())), preferred_element_type=jnp.float32)
    s = jax.nn.sigmoid(logits)
    sc = s + bias
    e_iota = jax.lax.broadcasted_iota(jnp.int32, sc.shape, 1)
    m0 = jnp.max(sc, axis=1, keepdims=True)
    i0 = jnp.min(jnp.where(sc == m0, e_iota, NUM_EXPERTS), axis=1, keepdims=True)
    oh0 = (e_iota == i0)
    sc2 = jnp.where(oh0, _NEG, sc)
    m1 = jnp.max(sc2, axis=1, keepdims=True)
    i1 = jnp.min(jnp.where(sc2 == m1, e_iota, NUM_EXPERTS), axis=1, keepdims=True)
    oh1 = (e_iota == i1)
    w0 = jnp.sum(jnp.where(oh0, s, 0.0), axis=1, keepdims=True)
    w1 = jnp.sum(jnp.where(oh1, s, 0.0), axis=1, keepdims=True)
    norm = w0 + w1 + 1e-6
    return (jnp.where(oh0, s, 0.0) + jnp.where(oh1, s, 0.0)) / norm


def _moe_body(hidden_ref, gate_w_ref, bias_ref, gup_ref, down_ref,
              out_ref, w_ref):
    e = pl.program_id(0)

    @pl.when(e == 0)
    def _do_route():
        w_ref[...] = _route(hidden_ref[...], gate_w_ref[...], bias_ref[...])

    wmat = w_ref[...]
    e_iota = jax.lax.broadcasted_iota(jnp.int32, wmat.shape, 1)
    wcol = jnp.sum(jnp.where(e_iota == e, wmat, 0.0), axis=1, keepdims=True)

    gup_b = gup_ref[0].astype(jnp.bfloat16)
    down_b = down_ref[0].astype(jnp.bfloat16)

    nchunk = 2
    cs = TOKENS // nchunk
    for c in range(nchunk):
        rows = pl.ds(c * cs, cs)
        x = hidden_ref[rows, :].astype(jnp.bfloat16)
        gu = jax.lax.dot_general(
            x, gup_b, (((1,), (1,)), ((), ())),
            preferred_element_type=jnp.float32)
        gate = gu[:, :INTER]
        up = gu[:, INTER:]
        act = ((gate * jax.nn.sigmoid(gate)) * up
               * wcol[c * cs:(c + 1) * cs, :]).astype(jnp.bfloat16)
        eo = jax.lax.dot_general(
            act, down_b, (((1,), (1,)), ((), ())),
            preferred_element_type=jnp.float32)

        @pl.when(e == 0)
        def _init():
            out_ref[rows, :] = eo

        @pl.when(e > 0)
        def _acc():
            out_ref[rows, :] += eo


@jax.jit
def kernel(hidden_states, gate_w, expert_bias, gate_up_proj, down_proj):
    out = pl.pallas_call(
        _moe_body,
        grid=(NUM_EXPERTS,),
        in_specs=[
            pl.BlockSpec((TOKENS, HIDDEN), lambda e: (0, 0)),
            pl.BlockSpec((NUM_EXPERTS, HIDDEN), lambda e: (0, 0)),
            pl.BlockSpec((1, NUM_EXPERTS), lambda e: (0, 0)),
            pl.BlockSpec((1, 2 * INTER, HIDDEN), lambda e: (e, 0, 0)),
            pl.BlockSpec((1, HIDDEN, INTER), lambda e: (e, 0, 0)),
        ],
        out_specs=pl.BlockSpec((TOKENS, HIDDEN), lambda e: (0, 0)),
        out_shape=jax.ShapeDtypeStruct((TOKENS, HIDDEN), jnp.float32),
        scratch_shapes=[pltpu.VMEM((TOKENS, NUM_EXPERTS), jnp.float32)],
    )(hidden_states, gate_w, expert_bias.reshape(1, NUM_EXPERTS),
      gate_up_proj, down_proj)
    return out


# speedup vs baseline: 1.0247x; 1.0247x over previous
import jax
import jax.numpy as jnp
from jax.experimental import pallas as pl
from jax.experimental.pallas import tpu as pltpu

NUM_EXPERTS = 8
TOP_K = 2
HIDDEN = 1024
INTER = 512
TOKENS = 2048

_NEG = -1e30


def _route(x, gw, bias):
    logits = jax.lax.dot_general(
        x, gw, (((1,), (1,)), ((), ())), preferred_element_type=jnp.float32)
    s = jax.nn.sigmoid(logits)
    sc = s + bias
    e_iota = jax.lax.broadcasted_iota(jnp.int32, sc.shape, 1)
    m0 = jnp.max(sc, axis=1, keepdims=True)
    i0 = jnp.min(jnp.where(sc == m0, e_iota, NUM_EXPERTS), axis=1, keepdims=True)
    oh0 = (e_iota == i0)
    sc2 = jnp.where(oh0, _NEG, sc)
    m1 = jnp.max(sc2, axis=1, keepdims=True)
    i1 = jnp.min(jnp.where(sc2 == m1, e_iota, NUM_EXPERTS), axis=1, keepdims=True)
    oh1 = (e_iota == i1)
    w0 = jnp.sum(jnp.where(oh0, s, 0.0), axis=1, keepdims=True)
    w1 = jnp.sum(jnp.where(oh1, s, 0.0), axis=1, keepdims=True)
    norm = w0 + w1 + 1e-6
    return (jnp.where(oh0, s, 0.0) + jnp.where(oh1, s, 0.0)) / norm


def _moe_body(hidden_ref, gate_w_ref, bias_ref, gup_ref, down_ref,
              out_ref, w_ref, act_ref):
    e = pl.program_id(0)

    @pl.when(e == 0)
    def _do_route():
        w_ref[...] = _route(hidden_ref[...], gate_w_ref[...], bias_ref[...])

    act_prev = act_ref[...]
    eo = jax.lax.dot_general(
        act_prev, down_ref[0].astype(jnp.bfloat16), (((1,), (1,)), ((), ())),
        preferred_element_type=jnp.float32)
    out_ref[...] = jnp.where(e >= 2, out_ref[...], 0.0) + eo

    wmat = w_ref[...]
    e_iota = jax.lax.broadcasted_iota(jnp.int32, wmat.shape, 1)
    wcol = jnp.sum(jnp.where(e_iota == e, wmat, 0.0), axis=1, keepdims=True)
    x = hidden_ref[...].astype(jnp.bfloat16)
    gu = jax.lax.dot_general(
        x, gup_ref[0].astype(jnp.bfloat16), (((1,), (1,)), ((), ())),
        preferred_element_type=jnp.float32)
    gate = gu[:, :INTER]
    up = gu[:, INTER:]
    act_ref[...] = ((gate * jax.nn.sigmoid(gate)) * up * wcol).astype(
        jnp.bfloat16)


@jax.jit
def kernel(hidden_states, gate_w, expert_bias, gate_up_proj, down_proj):
    out = pl.pallas_call(
        _moe_body,
        grid=(NUM_EXPERTS + 1,),
        in_specs=[
            pl.BlockSpec((TOKENS, HIDDEN), lambda e: (0, 0)),
            pl.BlockSpec((NUM_EXPERTS, HIDDEN), lambda e: (0, 0)),
            pl.BlockSpec((1, NUM_EXPERTS), lambda e: (0, 0)),
            pl.BlockSpec((1, 2 * INTER, HIDDEN),
                         lambda e: (jnp.minimum(e, NUM_EXPERTS - 1), 0, 0)),
            pl.BlockSpec((1, HIDDEN, INTER),
                         lambda e: (jnp.maximum(e - 1, 0), 0, 0)),
        ],
        out_specs=pl.BlockSpec((TOKENS, HIDDEN), lambda e: (0, 0)),
        out_shape=jax.ShapeDtypeStruct((TOKENS, HIDDEN), jnp.float32),
        scratch_shapes=[
            pltpu.VMEM((TOKENS, NUM_EXPERTS), jnp.float32),
            pltpu.VMEM((TOKENS, INTER), jnp.bfloat16),
        ],
    )(hidden_states, gate_w, expert_bias.reshape(1, NUM_EXPERTS),
      gate_up_proj, down_proj)
    return out
